# Initial kernel scaffold; baseline (speedup 1.0000x reference)
import functools

import jax
import jax.numpy as jnp
import numpy as np
from jax import lax
from jax.experimental import pallas as pl
from jax.experimental.pallas import tpu as pltpu
from jax.experimental.pallas import tpu_sc as plsc

NC, NS, LANES = 2, 16, 16
NW = NC * NS


def _build_time_mats(wr, wi, T):
    tp = np.arange(T)[:, None].astype(np.float32)
    t = np.arange(T)[None, :].astype(np.float32)
    th = np.pi * (t - tp) / 2.0
    c = jnp.asarray(np.cos(th))
    s = jnp.asarray(np.sin(th))
    sign = jnp.asarray(((-1.0) ** (t + tp)).astype(np.float32))
    A = (wr[None, None, :, :, 0]
         + 2.0 * (c[:, :, None, None] * wr[None, None, :, :, 1]
                  - s[:, :, None, None] * wi[None, None, :, :, 1])
         + sign[:, :, None, None] * wr[None, None, :, :, 2]) / 4.0
    Ti, _, Cin, Cout = A.shape
    return jnp.transpose(A, (0, 2, 1, 3)).reshape(Ti * Cin, Ti * Cout)


def _prelude_body(h_ref, acat_ref, w1a_ref, w1b_ref, v_ref, bv_ref,
                  h2_ref, p_ref, q_ref, vn_ref):
    T = h_ref.shape[0]
    C = h_ref.shape[2]
    hcat = jnp.concatenate([h_ref[t] for t in range(T)], axis=-1)
    xh = jnp.dot(hcat, acat_ref[...], preferred_element_type=jnp.float32)
    h2 = hcat + jnp.where(xh > 0, xh, 0.2 * xh)
    h2_ref[...] = h2
    for t in range(T):
        blk = h2[:, t * C:(t + 1) * C]
        p_ref[t] = jnp.dot(blk, w1a_ref[...], preferred_element_type=jnp.float32)
        q_ref[t] = jnp.dot(blk, w1b_ref[...], preferred_element_type=jnp.float32)
    v = v_ref[...]
    vn_ref[...] = v + jnp.dot(v, bv_ref[...], preferred_element_type=jnp.float32)


def _prelude(h, acat, w1a, w1b, v, bv, T, BN, C, BR):
    nblk = BN // BR
    return pl.pallas_call(
        _prelude_body,
        grid=(nblk,),
        in_specs=[
            pl.BlockSpec((T, BR, C), lambda i: (0, i, 0)),
            pl.BlockSpec((T * C, T * C), lambda i: (0, 0)),
            pl.BlockSpec((C, C), lambda i: (0, 0)),
            pl.BlockSpec((C, C), lambda i: (0, 0)),
            pl.BlockSpec((BR, 3 * T), lambda i: (i, 0)),
            pl.BlockSpec((3 * T, 3 * T), lambda i: (0, 0)),
        ],
        out_specs=[
            pl.BlockSpec((BR, T * C), lambda i: (i, 0)),
            pl.BlockSpec((T, BR, C), lambda i: (0, i, 0)),
            pl.BlockSpec((T, BR, C), lambda i: (0, i, 0)),
            pl.BlockSpec((BR, 3 * T), lambda i: (i, 0)),
        ],
        out_shape=[
            jax.ShapeDtypeStruct((BN, T * C), jnp.float32),
            jax.ShapeDtypeStruct((T, BN, C), jnp.float32),
            jax.ShapeDtypeStruct((T, BN, C), jnp.float32),
            jax.ShapeDtypeStruct((BN, 3 * T), jnp.float32),
        ],
    )(h, acat, w1a, w1b, v, bv)


def _d2_kernel(E, NV):
    per_w = E // NW
    ngrp = (per_w + LANES - 1) // LANES
    buf = ngrp * LANES

    @functools.partial(
        pl.kernel,
        out_type=jax.ShapeDtypeStruct((E,), jnp.float32),
        mesh=plsc.VectorSubcoreMesh(core_axis_name="c", subcore_axis_name="s"),
        scratch_types=[
            pltpu.VMEM((NV, 3), jnp.float32),
            pltpu.VMEM((buf,), jnp.int32),
            pltpu.VMEM((buf,), jnp.int32),
            pltpu.VMEM((buf,), jnp.float32),
        ],
    )
    def k(x_hbm, src_hbm, dst_hbm, d2_hbm, xv, sbuf, dbuf, obuf):
        wid = lax.axis_index("c") * NS + lax.axis_index("s")
        base = wid * per_w
        pltpu.sync_copy(x_hbm, xv)
        pltpu.sync_copy(src_hbm.at[pl.ds(base, per_w)], sbuf.at[pl.ds(0, per_w)])
        pltpu.sync_copy(dst_hbm.at[pl.ds(base, per_w)], dbuf.at[pl.ds(0, per_w)])
        lanes = lax.iota(jnp.int32, (LANES,))

        def body(j, _):
            off = j * LANES
            mask = (lanes + off) < per_w
            srcv = sbuf[pl.ds(off, LANES)]
            dstv = dbuf[pl.ds(off, LANES)]
            acc = jnp.zeros((LANES,), jnp.float32)
            for c in range(3):
                colc = jnp.full((LANES,), c, jnp.int32)
                xs = plsc.load_gather(xv, [srcv, colc], mask=mask)
                xd = plsc.load_gather(xv, [dstv, colc], mask=mask)
                d = xs - xd
                acc = acc + d * d
            obuf[pl.ds(off, LANES)] = acc
            return 0

        lax.fori_loop(0, ngrp, body, 0)
        pltpu.sync_copy(obuf.at[pl.ds(0, per_w)], d2_hbm.at[pl.ds(base, per_w)])

    return k


def _gather_kernel(TE, C, CE):
    per_w = TE // NW
    nch = per_w // CE

    @functools.partial(
        pl.kernel,
        out_type=[jax.ShapeDtypeStruct((TE, C), jnp.float32),
                  jax.ShapeDtypeStruct((TE, C), jnp.float32)],
        mesh=plsc.VectorSubcoreMesh(core_axis_name="c", subcore_axis_name="s"),
        scratch_types=[
            pltpu.VMEM((CE,), jnp.int32),
            pltpu.VMEM((CE,), jnp.int32),
            pltpu.VMEM((CE, C), jnp.float32),
            pltpu.VMEM((CE, C), jnp.float32),
            pltpu.SemaphoreType.DMA,
            pltpu.SemaphoreType.DMA,
        ],
    )
    def k(ptab, qtab, src_hbm, dst_hbm, gp_hbm, gq_hbm,
          sidx, didx, gpb, gqb, sem1, sem2):
        wid = lax.axis_index("c") * NS + lax.axis_index("s")
        wbase = wid * per_w

        def body(ci, _):
            base = wbase + ci * CE
            pltpu.sync_copy(src_hbm.at[pl.ds(base, CE)], sidx)
            pltpu.sync_copy(dst_hbm.at[pl.ds(base, CE)], didx)
            cp1 = pltpu.async_copy(ptab.at[sidx], gpb, sem1)
            cp2 = pltpu.async_copy(qtab.at[didx], gqb, sem2)
            cp1.wait()
            cp2.wait()
            pltpu.sync_copy(gpb, gp_hbm.at[pl.ds(base, CE)])
            pltpu.sync_copy(gqb, gq_hbm.at[pl.ds(base, CE)])
            return 0

        lax.fori_loop(0, nch, body, 0)

    return k


def _edge_body(gp_ref, gq_ref, d2_ref, w1c_ref, b1_ref, we2_ref, b2_ref,
               wg_ref, bg_ref, out_ref):
    g = (gp_ref[...] + gq_ref[...] + d2_ref[...] * w1c_ref[...] + b1_ref[...])
    m1 = g * jax.nn.sigmoid(g)
    m2 = jnp.dot(m1, we2_ref[...], preferred_element_type=jnp.float32) + b2_ref[...]
    m2 = m2 * jax.nn.sigmoid(m2)
    gate = jax.nn.sigmoid(
        jnp.sum(m2 * wg_ref[...] + bg_ref[...], axis=-1, keepdims=True))
    out_ref[...] = m2 * gate


def _edge_mlp(gp, gq, d2c, w1c, b1, we2, b2, wgr, bgr, TE, C, BE):
    nblk = TE // BE
    return pl.pallas_call(
        _edge_body,
        grid=(nblk,),
        in_specs=[
            pl.BlockSpec((BE, C), lambda i: (i, 0)),
            pl.BlockSpec((BE, C), lambda i: (i, 0)),
            pl.BlockSpec((BE, 1), lambda i: (i, 0)),
            pl.BlockSpec((1, C), lambda i: (0, 0)),
            pl.BlockSpec((1, C), lambda i: (0, 0)),
            pl.BlockSpec((C, C), lambda i: (0, 0)),
            pl.BlockSpec((1, C), lambda i: (0, 0)),
            pl.BlockSpec((1, C), lambda i: (0, 0)),
            pl.BlockSpec((1, C), lambda i: (0, 0)),
        ],
        out_specs=pl.BlockSpec((BE, C), lambda i: (i, 0)),
        out_shape=jax.ShapeDtypeStruct((TE, C), jnp.float32),
    )(gp, gq, d2c, w1c, b1, we2, b2, wgr, bgr)


def _scatter_kernel(T, BN, E, C, CE):
    per_tile = E // NS
    nch = per_tile // CE
    rows = BN // NS
    reps = T // NC

    @functools.partial(
        pl.kernel,
        out_type=jax.ShapeDtypeStruct((T * BN, C), jnp.float32),
        mesh=plsc.VectorSubcoreMesh(core_axis_name="c", subcore_axis_name="s"),
        scratch_types=[
            pltpu.VMEM_SHARED((BN, C), jnp.float32),
            pltpu.VMEM((CE, C), jnp.float32),
            pltpu.VMEM((nch, CE), jnp.int32),
        ],
    )
    def k(m_hbm, dst_hbm, zero_hbm, agg_hbm, aggS, mbuf, dbuf):
        cid = lax.axis_index("c")
        sid = lax.axis_index("s")
        ebase0 = sid * per_tile

        def load_idx(ci, _):
            pltpu.sync_copy(dst_hbm.at[pl.ds(ebase0 + ci * CE, CE)], dbuf.at[ci])
            return 0

        lax.fori_loop(0, nch, load_idx, 0)

        for rep in range(reps):
            t = cid + NC * rep
            pltpu.sync_copy(zero_hbm.at[pl.ds(sid * rows, rows)],
                            aggS.at[pl.ds(sid * rows, rows)])
            plsc.subcore_barrier()

            def body(ci, _):
                base = t * E + ebase0 + ci * CE
                pltpu.sync_copy(m_hbm.at[pl.ds(base, CE)], mbuf)
                pltpu.sync_copy(mbuf, aggS.at[dbuf.at[ci]], add=True)
                return 0

            lax.fori_loop(0, nch, body, 0)
            plsc.subcore_barrier()
            pltpu.sync_copy(aggS.at[pl.ds(sid * rows, rows)],
                            agg_hbm.at[pl.ds(t * BN + sid * rows, rows)])

    return k


def _node_body(h2_ref, agg_ref, wa_ref, wb_ref, b1_ref, w2_ref, b2_ref, out_ref):
    u = (jnp.dot(h2_ref[...], wa_ref[...], preferred_element_type=jnp.float32)
         + jnp.dot(agg_ref[...], wb_ref[...], preferred_element_type=jnp.float32)
         + b1_ref[...])
    u = u * jax.nn.sigmoid(u)
    out_ref[...] = (h2_ref[...]
                    + jnp.dot(u, w2_ref[...], preferred_element_type=jnp.float32)
                    + b2_ref[...])


def _node_update(h2cat, agg, wa, wb, b1, w2, b2, T, BN, C, BR):
    nblk = BN // BR
    return pl.pallas_call(
        _node_body,
        grid=(T, nblk),
        in_specs=[
            pl.BlockSpec((BR, C), lambda t, i: (i, t)),
            pl.BlockSpec((BR, C), lambda t, i: (t * nblk + i, 0)),
            pl.BlockSpec((C, C), lambda t, i: (0, 0)),
            pl.BlockSpec((C, C), lambda t, i: (0, 0)),
            pl.BlockSpec((1, C), lambda t, i: (0, 0)),
            pl.BlockSpec((C, C), lambda t, i: (0, 0)),
            pl.BlockSpec((1, C), lambda t, i: (0, 0)),
        ],
        out_specs=pl.BlockSpec((BR, C), lambda t, i: (t * nblk + i, 0)),
        out_shape=jax.ShapeDtypeStruct((T * BN, C), jnp.float32),
    )(h2cat, agg, wa, wb, b1, w2, b2)


def kernel(h, x, vel_all, edge_index, tc_h_wr, tc_h_wi, tc_v_wr, tc_v_wi,
           W_e1, b_e1, W_e2, b_e2, W_g, b_g, W_n1, b_n1, W_n2, b_n2):
    T, BN, C = h.shape
    E = edge_index.shape[1]
    TE = T * E

    acat = _build_time_mats(tc_h_wr, tc_h_wi, T)
    a_v = _build_time_mats(tc_v_wr, tc_v_wi, T)
    bv = jnp.kron(a_v, jnp.eye(3, dtype=jnp.float32))
    w1a, w1b = W_e1[:C], W_e1[C:2 * C]
    w1c = W_e1[2 * C].reshape(1, C)
    src0 = edge_index[0].astype(jnp.int32)
    dst0 = edge_index[1].astype(jnp.int32)

    vflat = vel_all.reshape(BN, T * 3)
    h2cat, ptab, qtab, vnew = _prelude(h, acat, w1a, w1b, vflat, bv,
                                       T, BN, C, BR=1000)
    vel_out = vnew.reshape(BN, T, 3)

    d2 = _d2_kernel(E, BN)(x, src0, dst0)

    offs = jnp.repeat(jnp.arange(T, dtype=jnp.int32) * BN, E)
    src_all = jnp.tile(src0, T) + offs
    dst_all = jnp.tile(dst0, T) + offs
    gp, gq = _gather_kernel(TE, C, CE=400)(
        ptab.reshape(T * BN, C), qtab.reshape(T * BN, C), src_all, dst_all)

    d2c = jnp.tile(d2, T).reshape(TE, 1)
    bgr = jnp.full((1, C), b_g[0] / C, jnp.float32)
    m = _edge_mlp(gp, gq, d2c, w1c, b_e1.reshape(1, C), W_e2,
                  b_e2.reshape(1, C), W_g.reshape(1, C), bgr, TE, C, BE=2000)

    zeros = jnp.zeros((BN, C), jnp.float32)
    agg = _scatter_kernel(T, BN, E, C, CE=400)(m, dst0, zeros)

    h_out = _node_update(h2cat, agg, W_n1[:C], W_n1[C:], b_n1.reshape(1, C),
                         W_n2, b_n2.reshape(1, C), T, BN, C, BR=1000)
    return h_out.reshape(T, BN, C), vel_out


# speedup vs baseline: 7.2066x; 7.2066x over previous
import functools

import jax
import jax.numpy as jnp
import numpy as np
from jax import lax
from jax.experimental import pallas as pl
from jax.experimental.pallas import tpu as pltpu
from jax.experimental.pallas import tpu_sc as plsc

NC, NS, LANES = 2, 16, 16
NW = NC * NS


def _build_time_mats(wr, wi, T):
    tp = np.arange(T)[:, None].astype(np.float32)
    t = np.arange(T)[None, :].astype(np.float32)
    th = np.pi * (t - tp) / 2.0
    c = jnp.asarray(np.cos(th))
    s = jnp.asarray(np.sin(th))
    sign = jnp.asarray(((-1.0) ** (t + tp)).astype(np.float32))
    A = (wr[None, None, :, :, 0]
         + 2.0 * (c[:, :, None, None] * wr[None, None, :, :, 1]
                  - s[:, :, None, None] * wi[None, None, :, :, 1])
         + sign[:, :, None, None] * wr[None, None, :, :, 2]) / 4.0
    Ti, _, Cin, Cout = A.shape
    return jnp.transpose(A, (0, 2, 1, 3)).reshape(Ti * Cin, Ti * Cout)


def _prelude_body(h_ref, acat_ref, w1a_ref, w1b_ref, v_ref, bv_ref,
                  h2_ref, p_ref, q_ref, vn_ref):
    T = h_ref.shape[0]
    C = h_ref.shape[2]
    hcat = jnp.concatenate([h_ref[t] for t in range(T)], axis=-1)
    xh = jnp.dot(hcat, acat_ref[...], preferred_element_type=jnp.float32)
    h2 = hcat + jnp.where(xh > 0, xh, 0.2 * xh)
    h2_ref[...] = h2
    for t in range(T):
        blk = h2[:, t * C:(t + 1) * C]
        p_ref[t] = jnp.dot(blk, w1a_ref[...], preferred_element_type=jnp.float32)
        q_ref[t] = jnp.dot(blk, w1b_ref[...], preferred_element_type=jnp.float32)
    v = v_ref[...]
    vn_ref[...] = v + jnp.dot(v, bv_ref[...], preferred_element_type=jnp.float32)


def _prelude(h, acat, w1a, w1b, v, bv, T, BN, C, BR):
    nblk = BN // BR
    return pl.pallas_call(
        _prelude_body,
        grid=(nblk,),
        in_specs=[
            pl.BlockSpec((T, BR, C), lambda i: (0, i, 0)),
            pl.BlockSpec((T * C, T * C), lambda i: (0, 0)),
            pl.BlockSpec((C, C), lambda i: (0, 0)),
            pl.BlockSpec((C, C), lambda i: (0, 0)),
            pl.BlockSpec((BR, 3 * T), lambda i: (i, 0)),
            pl.BlockSpec((3 * T, 3 * T), lambda i: (0, 0)),
        ],
        out_specs=[
            pl.BlockSpec((BR, T * C), lambda i: (i, 0)),
            pl.BlockSpec((T, BR, C), lambda i: (0, i, 0)),
            pl.BlockSpec((T, BR, C), lambda i: (0, i, 0)),
            pl.BlockSpec((BR, 3 * T), lambda i: (i, 0)),
        ],
        out_shape=[
            jax.ShapeDtypeStruct((BN, T * C), jnp.float32),
            jax.ShapeDtypeStruct((T, BN, C), jnp.float32),
            jax.ShapeDtypeStruct((T, BN, C), jnp.float32),
            jax.ShapeDtypeStruct((BN, 3 * T), jnp.float32),
        ],
    )(h, acat, w1a, w1b, v, bv)


def _gather_kernel(TE, C, CE):
    per_w = TE // NW
    nch = per_w // CE

    @functools.partial(
        pl.kernel,
        out_type=[jax.ShapeDtypeStruct((TE, C), jnp.float32),
                  jax.ShapeDtypeStruct((TE, C), jnp.float32)],
        mesh=plsc.VectorSubcoreMesh(core_axis_name="c", subcore_axis_name="s"),
        scratch_types=[
            pltpu.VMEM((CE,), jnp.int32),
            pltpu.VMEM((CE,), jnp.int32),
            pltpu.VMEM((CE, C), jnp.float32),
            pltpu.VMEM((CE, C), jnp.float32),
            pltpu.SemaphoreType.DMA,
            pltpu.SemaphoreType.DMA,
        ],
    )
    def k(ptab, qtab, src_hbm, dst_hbm, gp_hbm, gq_hbm,
          sidx, didx, gpb, gqb, sem1, sem2):
        wid = lax.axis_index("c") * NS + lax.axis_index("s")
        wbase = wid * per_w

        def body(ci, _):
            base = wbase + ci * CE
            pltpu.sync_copy(src_hbm.at[pl.ds(base, CE)], sidx)
            pltpu.sync_copy(dst_hbm.at[pl.ds(base, CE)], didx)
            cp1 = pltpu.async_copy(ptab.at[sidx], gpb, sem1)
            cp2 = pltpu.async_copy(qtab.at[didx], gqb, sem2)
            cp1.wait()
            cp2.wait()
            pltpu.sync_copy(gpb, gp_hbm.at[pl.ds(base, CE)])
            pltpu.sync_copy(gqb, gq_hbm.at[pl.ds(base, CE)])
            return 0

        lax.fori_loop(0, nch, body, 0)

    return k


def _edge_body(gp_ref, gq_ref, xs_ref, xd_ref, w1c_ref, b1_ref, we2_ref, b2_ref,
               wg_ref, bg_ref, out_ref):
    rel = xs_ref[...] - xd_ref[...]
    mask3 = lax.broadcasted_iota(jnp.int32, (1, rel.shape[1]), 1) < 3
    d2 = jnp.sum(jnp.where(mask3, rel * rel, 0.0), axis=-1, keepdims=True)
    g = (gp_ref[...] + gq_ref[...] + d2 * w1c_ref[...] + b1_ref[...])
    m1 = g * jax.nn.sigmoid(g)
    m2 = jnp.dot(m1, we2_ref[...], preferred_element_type=jnp.float32) + b2_ref[...]
    m2 = m2 * jax.nn.sigmoid(m2)
    gate = jax.nn.sigmoid(
        jnp.sum(m2 * wg_ref[...] + bg_ref[...], axis=-1, keepdims=True))
    out_ref[...] = m2 * gate


def _edge_mlp(gp, gq, xs, xd, w1c, b1, we2, b2, wgr, bgr, T, E, C, XW, BE):
    nblk = E // BE
    return pl.pallas_call(
        _edge_body,
        grid=(T, nblk),
        in_specs=[
            pl.BlockSpec((BE, C), lambda t, i: (t * nblk + i, 0)),
            pl.BlockSpec((BE, C), lambda t, i: (t * nblk + i, 0)),
            pl.BlockSpec((BE, XW), lambda t, i: (i, 0)),
            pl.BlockSpec((BE, XW), lambda t, i: (i, 0)),
            pl.BlockSpec((1, C), lambda t, i: (0, 0)),
            pl.BlockSpec((1, C), lambda t, i: (0, 0)),
            pl.BlockSpec((C, C), lambda t, i: (0, 0)),
            pl.BlockSpec((1, C), lambda t, i: (0, 0)),
            pl.BlockSpec((1, C), lambda t, i: (0, 0)),
            pl.BlockSpec((1, C), lambda t, i: (0, 0)),
        ],
        out_specs=pl.BlockSpec((BE, C), lambda t, i: (t * nblk + i, 0)),
        out_shape=jax.ShapeDtypeStruct((T * E, C), jnp.float32),
    )(gp, gq, xs, xd, w1c, b1, we2, b2, wgr, bgr)


def _scatter_kernel(T, BN, E, C, CE):
    per_tile = E // NS
    nch = per_tile // CE
    rows = (BN // NS) // 8 * 8
    tail = BN - rows * NS
    reps = T // NC

    @functools.partial(
        pl.kernel,
        out_type=jax.ShapeDtypeStruct((T * BN, C), jnp.float32),
        mesh=plsc.VectorSubcoreMesh(core_axis_name="c", subcore_axis_name="s"),
        scratch_types=[
            pltpu.VMEM_SHARED((BN, C), jnp.float32),
            pltpu.VMEM((CE, C), jnp.float32),
            pltpu.VMEM((CE,), jnp.int32),
        ],
    )
    def k(m_hbm, dst_hbm, zero_hbm, agg_hbm, aggS, mbuf, dbuf):
        cid = lax.axis_index("c")
        sid = lax.axis_index("s")
        ebase0 = sid * per_tile

        for rep in range(reps):
            t = cid + NC * rep
            pltpu.sync_copy(zero_hbm.at[pl.ds(sid * rows, rows)],
                            aggS.at[pl.ds(sid * rows, rows)])
            if tail:
                @pl.when(sid == 0)
                def _():
                    pltpu.sync_copy(zero_hbm.at[pl.ds(NS * rows, tail)],
                                    aggS.at[pl.ds(NS * rows, tail)])
            plsc.subcore_barrier()

            def body(ci, _):
                base = t * E + ebase0 + ci * CE
                pltpu.sync_copy(m_hbm.at[pl.ds(base, CE)], mbuf)
                pltpu.sync_copy(dst_hbm.at[pl.ds(ebase0 + ci * CE, CE)], dbuf)
                pltpu.sync_copy(mbuf, aggS.at[dbuf], add=True)
                return 0

            lax.fori_loop(0, nch, body, 0)
            plsc.subcore_barrier()
            pltpu.sync_copy(aggS.at[pl.ds(sid * rows, rows)],
                            agg_hbm.at[pl.ds(t * BN + sid * rows, rows)])
            if tail:
                @pl.when(sid == 0)
                def _():
                    pltpu.sync_copy(aggS.at[pl.ds(NS * rows, tail)],
                                    agg_hbm.at[pl.ds(t * BN + NS * rows, tail)])

    return k


def _node_body(h2_ref, agg_ref, wa_ref, wb_ref, b1_ref, w2_ref, b2_ref, out_ref):
    u = (jnp.dot(h2_ref[...], wa_ref[...], preferred_element_type=jnp.float32)
         + jnp.dot(agg_ref[...], wb_ref[...], preferred_element_type=jnp.float32)
         + b1_ref[...])
    u = u * jax.nn.sigmoid(u)
    out_ref[...] = (h2_ref[...]
                    + jnp.dot(u, w2_ref[...], preferred_element_type=jnp.float32)
                    + b2_ref[...])


def _node_update(h2cat, agg, wa, wb, b1, w2, b2, T, BN, C, BR):
    nblk = BN // BR
    return pl.pallas_call(
        _node_body,
        grid=(T, nblk),
        in_specs=[
            pl.BlockSpec((BR, C), lambda t, i: (i, t)),
            pl.BlockSpec((BR, C), lambda t, i: (t * nblk + i, 0)),
            pl.BlockSpec((C, C), lambda t, i: (0, 0)),
            pl.BlockSpec((C, C), lambda t, i: (0, 0)),
            pl.BlockSpec((1, C), lambda t, i: (0, 0)),
            pl.BlockSpec((C, C), lambda t, i: (0, 0)),
            pl.BlockSpec((1, C), lambda t, i: (0, 0)),
        ],
        out_specs=pl.BlockSpec((BR, C), lambda t, i: (t * nblk + i, 0)),
        out_shape=jax.ShapeDtypeStruct((T * BN, C), jnp.float32),
    )(h2cat, agg, wa, wb, b1, w2, b2)


def kernel(h, x, vel_all, edge_index, tc_h_wr, tc_h_wi, tc_v_wr, tc_v_wi,
           W_e1, b_e1, W_e2, b_e2, W_g, b_g, W_n1, b_n1, W_n2, b_n2):
    T, BN, C = h.shape
    E = edge_index.shape[1]
    TE = T * E

    acat = _build_time_mats(tc_h_wr, tc_h_wi, T)
    a_v = _build_time_mats(tc_v_wr, tc_v_wi, T)
    bv = jnp.kron(a_v, jnp.eye(3, dtype=jnp.float32))
    w1a, w1b = W_e1[:C], W_e1[C:2 * C]
    w1c = W_e1[2 * C].reshape(1, C)
    src0 = edge_index[0].astype(jnp.int32)
    dst0 = edge_index[1].astype(jnp.int32)

    vflat = vel_all.reshape(BN, T * 3)
    h2cat, ptab, qtab, vnew = _prelude(h, acat, w1a, w1b, vflat, bv,
                                       T, BN, C, BR=1000)
    vel_out = vnew.reshape(BN, T, 3)

    XW = 128
    x16 = jnp.zeros((BN, XW), jnp.float32).at[:, :3].set(x)
    xs_g, xd_g = _gather_kernel(E, XW, CE=200)(x16, x16, src0, dst0)

    offs = jnp.repeat(jnp.arange(T, dtype=jnp.int32) * BN, E)
    src_all = jnp.tile(src0, T) + offs
    dst_all = jnp.tile(dst0, T) + offs
    gp, gq = _gather_kernel(TE, C, CE=400)(
        ptab.reshape(T * BN, C), qtab.reshape(T * BN, C), src_all, dst_all)

    bgr = jnp.full((1, C), b_g[0] / C, jnp.float32)
    m = _edge_mlp(gp, gq, xs_g, xd_g, w1c, b_e1.reshape(1, C), W_e2,
                  b_e2.reshape(1, C), W_g.reshape(1, C), bgr, T, E, C, XW, BE=2000)

    zeros = jnp.zeros((BN, C), jnp.float32)
    agg = _scatter_kernel(T, BN, E, C, CE=200)(m, dst0, zeros)

    h_out = _node_update(h2cat, agg, W_n1[:C], W_n1[C:], b_n1.reshape(1, C),
                         W_n2, b_n2.reshape(1, C), T, BN, C, BR=1000)
    return h_out.reshape(T, BN, C), vel_out


# speedup vs baseline: 7.8255x; 1.0859x over previous
import functools

import jax
import jax.numpy as jnp
import numpy as np
from jax import lax
from jax.experimental import pallas as pl
from jax.experimental.pallas import tpu as pltpu
from jax.experimental.pallas import tpu_sc as plsc

NC, NS, LANES = 2, 16, 16
NW = NC * NS


def _build_time_mats(wr, wi, T):
    tp = np.arange(T)[:, None].astype(np.float32)
    t = np.arange(T)[None, :].astype(np.float32)
    th = np.pi * (t - tp) / 2.0
    c = jnp.asarray(np.cos(th))
    s = jnp.asarray(np.sin(th))
    sign = jnp.asarray(((-1.0) ** (t + tp)).astype(np.float32))
    A = (wr[None, None, :, :, 0]
         + 2.0 * (c[:, :, None, None] * wr[None, None, :, :, 1]
                  - s[:, :, None, None] * wi[None, None, :, :, 1])
         + sign[:, :, None, None] * wr[None, None, :, :, 2]) / 4.0
    Ti, _, Cin, Cout = A.shape
    return jnp.transpose(A, (0, 2, 1, 3)).reshape(Ti * Cin, Ti * Cout)


def _prelude_body(h_ref, acat_ref, w1a_ref, w1b_ref, v_ref, bv_ref,
                  h2_ref, p_ref, q_ref, vn_ref):
    T = h_ref.shape[0]
    C = h_ref.shape[2]
    hcat = jnp.concatenate([h_ref[t] for t in range(T)], axis=-1)
    xh = jnp.dot(hcat, acat_ref[...], preferred_element_type=jnp.float32)
    h2 = hcat + jnp.where(xh > 0, xh, 0.2 * xh)
    h2_ref[...] = h2
    for t in range(T):
        blk = h2[:, t * C:(t + 1) * C]
        p_ref[t] = jnp.dot(blk, w1a_ref[...], preferred_element_type=jnp.float32)
        q_ref[t] = jnp.dot(blk, w1b_ref[...], preferred_element_type=jnp.float32)
    v = v_ref[...]
    vn_ref[...] = v + jnp.dot(v, bv_ref[...], preferred_element_type=jnp.float32)


def _prelude(h, acat, w1a, w1b, v, bv, T, BN, C, BR):
    nblk = BN // BR
    return pl.pallas_call(
        _prelude_body,
        grid=(nblk,),
        in_specs=[
            pl.BlockSpec((T, BR, C), lambda i: (0, i, 0)),
            pl.BlockSpec((T * C, T * C), lambda i: (0, 0)),
            pl.BlockSpec((C, C), lambda i: (0, 0)),
            pl.BlockSpec((C, C), lambda i: (0, 0)),
            pl.BlockSpec((BR, 3 * T), lambda i: (i, 0)),
            pl.BlockSpec((3 * T, 3 * T), lambda i: (0, 0)),
        ],
        out_specs=[
            pl.BlockSpec((BR, T * C), lambda i: (i, 0)),
            pl.BlockSpec((T, BR, C), lambda i: (0, i, 0)),
            pl.BlockSpec((T, BR, C), lambda i: (0, i, 0)),
            pl.BlockSpec((BR, 3 * T), lambda i: (i, 0)),
        ],
        out_shape=[
            jax.ShapeDtypeStruct((BN, T * C), jnp.float32),
            jax.ShapeDtypeStruct((T, BN, C), jnp.float32),
            jax.ShapeDtypeStruct((T, BN, C), jnp.float32),
            jax.ShapeDtypeStruct((BN, 3 * T), jnp.float32),
        ],
    )(h, acat, w1a, w1b, v, bv)


def _gather_kernel(TE, C, CE):
    per_w = TE // NW
    nch = per_w // CE

    @functools.partial(
        pl.kernel,
        out_type=[jax.ShapeDtypeStruct((TE, C), jnp.float32),
                  jax.ShapeDtypeStruct((TE, C), jnp.float32)],
        mesh=plsc.VectorSubcoreMesh(core_axis_name="c", subcore_axis_name="s"),
        scratch_types=[
            pltpu.VMEM((CE,), jnp.int32),
            pltpu.VMEM((CE,), jnp.int32),
            pltpu.VMEM((CE, C), jnp.float32),
            pltpu.VMEM((CE, C), jnp.float32),
            pltpu.SemaphoreType.DMA,
            pltpu.SemaphoreType.DMA,
        ],
    )
    def k(ptab, qtab, src_hbm, dst_hbm, gp_hbm, gq_hbm,
          sidx, didx, gpb, gqb, sem1, sem2):
        wid = lax.axis_index("c") * NS + lax.axis_index("s")
        wbase = wid * per_w

        def body(ci, _):
            base = wbase + ci * CE
            pltpu.sync_copy(src_hbm.at[pl.ds(base, CE)], sidx)
            pltpu.sync_copy(dst_hbm.at[pl.ds(base, CE)], didx)
            cp1 = pltpu.async_copy(ptab.at[sidx], gpb, sem1)
            cp2 = pltpu.async_copy(qtab.at[didx], gqb, sem2)
            cp1.wait()
            cp2.wait()
            pltpu.sync_copy(gpb, gp_hbm.at[pl.ds(base, CE)])
            pltpu.sync_copy(gqb, gq_hbm.at[pl.ds(base, CE)])
            return 0

        lax.fori_loop(0, nch, body, 0)

    return k


def _gather_add_kernel(TE, C, CE):
    per_w = TE // NW
    nch = per_w // CE

    @functools.partial(
        pl.kernel,
        out_type=jax.ShapeDtypeStruct((TE, C), jnp.float32),
        mesh=plsc.VectorSubcoreMesh(core_axis_name="c", subcore_axis_name="s"),
        scratch_types=[
            pltpu.VMEM((CE,), jnp.int32),
            pltpu.VMEM((CE,), jnp.int32),
            pltpu.VMEM((CE, C), jnp.float32),
            pltpu.SemaphoreType.DMA,
        ],
    )
    def k(ptab, qtab, src_hbm, dst_hbm, g_hbm, sidx, didx, gb, sem):
        wid = lax.axis_index("c") * NS + lax.axis_index("s")
        wbase = wid * per_w

        def body(ci, _):
            base = wbase + ci * CE
            pltpu.sync_copy(src_hbm.at[pl.ds(base, CE)], sidx)
            pltpu.sync_copy(dst_hbm.at[pl.ds(base, CE)], didx)
            pltpu.async_copy(qtab.at[didx], gb, sem).wait()
            pltpu.async_copy(ptab.at[sidx], gb, sem, add=True).wait()
            pltpu.sync_copy(gb, g_hbm.at[pl.ds(base, CE)])
            return 0

        lax.fori_loop(0, nch, body, 0)

    return k


def _edge_body(g_ref, xs_ref, xd_ref, w1c_ref, b1_ref, we2_ref, b2_ref,
               wg_ref, bg_ref, out_ref):
    rel = xs_ref[...] - xd_ref[...]
    mask3 = lax.broadcasted_iota(jnp.int32, (1, rel.shape[1]), 1) < 3
    d2 = jnp.sum(jnp.where(mask3, rel * rel, 0.0), axis=-1, keepdims=True)
    g = (g_ref[...] + d2 * w1c_ref[...] + b1_ref[...])
    m1 = g * jax.nn.sigmoid(g)
    m2 = jnp.dot(m1, we2_ref[...], preferred_element_type=jnp.float32) + b2_ref[...]
    m2 = m2 * jax.nn.sigmoid(m2)
    gate = jax.nn.sigmoid(
        jnp.sum(m2 * wg_ref[...] + bg_ref[...], axis=-1, keepdims=True))
    out_ref[...] = m2 * gate


def _edge_mlp(g, xs, xd, w1c, b1, we2, b2, wgr, bgr, T, E, C, XW, BE):
    nblk = E // BE
    return pl.pallas_call(
        _edge_body,
        grid=(T, nblk),
        in_specs=[
            pl.BlockSpec((BE, C), lambda t, i: (t * nblk + i, 0)),
            pl.BlockSpec((BE, XW), lambda t, i: (i, 0)),
            pl.BlockSpec((BE, XW), lambda t, i: (i, 0)),
            pl.BlockSpec((1, C), lambda t, i: (0, 0)),
            pl.BlockSpec((1, C), lambda t, i: (0, 0)),
            pl.BlockSpec((C, C), lambda t, i: (0, 0)),
            pl.BlockSpec((1, C), lambda t, i: (0, 0)),
            pl.BlockSpec((1, C), lambda t, i: (0, 0)),
            pl.BlockSpec((1, C), lambda t, i: (0, 0)),
        ],
        out_specs=pl.BlockSpec((BE, C), lambda t, i: (t * nblk + i, 0)),
        out_shape=jax.ShapeDtypeStruct((T * E, C), jnp.float32),
    )(g, xs, xd, w1c, b1, we2, b2, wgr, bgr)


def _scatter_kernel(T, BN, E, C, CE):
    per_tile = E // NS
    nch = per_tile // CE
    rows = (BN // NS) // 8 * 8
    tail = BN - rows * NS
    reps = T // NC

    @functools.partial(
        pl.kernel,
        out_type=jax.ShapeDtypeStruct((T * BN, C), jnp.float32),
        mesh=plsc.VectorSubcoreMesh(core_axis_name="c", subcore_axis_name="s"),
        scratch_types=[
            pltpu.VMEM_SHARED((BN, C), jnp.float32),
            pltpu.VMEM((CE, C), jnp.float32),
            pltpu.VMEM((CE,), jnp.int32),
        ],
    )
    def k(m_hbm, dst_hbm, zero_hbm, agg_hbm, aggS, mbuf, dbuf):
        cid = lax.axis_index("c")
        sid = lax.axis_index("s")
        ebase0 = sid * per_tile

        for rep in range(reps):
            t = cid + NC * rep
            pltpu.sync_copy(zero_hbm.at[pl.ds(sid * rows, rows)],
                            aggS.at[pl.ds(sid * rows, rows)])
            if tail:
                @pl.when(sid == 0)
                def _():
                    pltpu.sync_copy(zero_hbm.at[pl.ds(NS * rows, tail)],
                                    aggS.at[pl.ds(NS * rows, tail)])
            plsc.subcore_barrier()

            def body(ci, _):
                base = t * E + ebase0 + ci * CE
                pltpu.sync_copy(m_hbm.at[pl.ds(base, CE)], mbuf)
                pltpu.sync_copy(dst_hbm.at[pl.ds(ebase0 + ci * CE, CE)], dbuf)
                pltpu.sync_copy(mbuf, aggS.at[dbuf], add=True)
                return 0

            lax.fori_loop(0, nch, body, 0)
            plsc.subcore_barrier()
            pltpu.sync_copy(aggS.at[pl.ds(sid * rows, rows)],
                            agg_hbm.at[pl.ds(t * BN + sid * rows, rows)])
            if tail:
                @pl.when(sid == 0)
                def _():
                    pltpu.sync_copy(aggS.at[pl.ds(NS * rows, tail)],
                                    agg_hbm.at[pl.ds(t * BN + NS * rows, tail)])

    return k


def _node_body(h2_ref, agg_ref, wa_ref, wb_ref, b1_ref, w2_ref, b2_ref, out_ref):
    u = (jnp.dot(h2_ref[...], wa_ref[...], preferred_element_type=jnp.float32)
         + jnp.dot(agg_ref[...], wb_ref[...], preferred_element_type=jnp.float32)
         + b1_ref[...])
    u = u * jax.nn.sigmoid(u)
    out_ref[...] = (h2_ref[...]
                    + jnp.dot(u, w2_ref[...], preferred_element_type=jnp.float32)
                    + b2_ref[...])


def _node_update(h2cat, agg, wa, wb, b1, w2, b2, T, BN, C, BR):
    nblk = BN // BR
    return pl.pallas_call(
        _node_body,
        grid=(T, nblk),
        in_specs=[
            pl.BlockSpec((BR, C), lambda t, i: (i, t)),
            pl.BlockSpec((BR, C), lambda t, i: (t * nblk + i, 0)),
            pl.BlockSpec((C, C), lambda t, i: (0, 0)),
            pl.BlockSpec((C, C), lambda t, i: (0, 0)),
            pl.BlockSpec((1, C), lambda t, i: (0, 0)),
            pl.BlockSpec((C, C), lambda t, i: (0, 0)),
            pl.BlockSpec((1, C), lambda t, i: (0, 0)),
        ],
        out_specs=pl.BlockSpec((BR, C), lambda t, i: (t * nblk + i, 0)),
        out_shape=jax.ShapeDtypeStruct((T * BN, C), jnp.float32),
    )(h2cat, agg, wa, wb, b1, w2, b2)


def kernel(h, x, vel_all, edge_index, tc_h_wr, tc_h_wi, tc_v_wr, tc_v_wi,
           W_e1, b_e1, W_e2, b_e2, W_g, b_g, W_n1, b_n1, W_n2, b_n2):
    T, BN, C = h.shape
    E = edge_index.shape[1]
    TE = T * E

    acat = _build_time_mats(tc_h_wr, tc_h_wi, T)
    a_v = _build_time_mats(tc_v_wr, tc_v_wi, T)
    bv = jnp.kron(a_v, jnp.eye(3, dtype=jnp.float32))
    w1a, w1b = W_e1[:C], W_e1[C:2 * C]
    w1c = W_e1[2 * C].reshape(1, C)
    src0 = edge_index[0].astype(jnp.int32)
    dst0 = edge_index[1].astype(jnp.int32)

    vflat = vel_all.reshape(BN, T * 3)
    h2cat, ptab, qtab, vnew = _prelude(h, acat, w1a, w1b, vflat, bv,
                                       T, BN, C, BR=1000)
    vel_out = vnew.reshape(BN, T, 3)

    XW = 128
    x16 = jnp.zeros((BN, XW), jnp.float32).at[:, :3].set(x)
    xs_g, xd_g = _gather_kernel(E, XW, CE=200)(x16, x16, src0, dst0)

    offs = jnp.repeat(jnp.arange(T, dtype=jnp.int32) * BN, E)
    src_all = jnp.tile(src0, T) + offs
    dst_all = jnp.tile(dst0, T) + offs
    g = _gather_add_kernel(TE, C, CE=400)(
        ptab.reshape(T * BN, C), qtab.reshape(T * BN, C), src_all, dst_all)

    bgr = jnp.full((1, C), b_g[0] / C, jnp.float32)
    m = _edge_mlp(g, xs_g, xd_g, w1c, b_e1.reshape(1, C), W_e2,
                  b_e2.reshape(1, C), W_g.reshape(1, C), bgr, T, E, C, XW, BE=2000)

    zeros = jnp.zeros((BN, C), jnp.float32)
    agg = _scatter_kernel(T, BN, E, C, CE=200)(m, dst0, zeros)

    h_out = _node_update(h2cat, agg, W_n1[:C], W_n1[C:], b_n1.reshape(1, C),
                         W_n2, b_n2.reshape(1, C), T, BN, C, BR=1000)
    return h_out.reshape(T, BN, C), vel_out


# speedup vs baseline: 8.3445x; 1.0663x over previous
import functools

import jax
import jax.numpy as jnp
import numpy as np
from jax import lax
from jax.experimental import pallas as pl
from jax.experimental.pallas import tpu as pltpu
from jax.experimental.pallas import tpu_sc as plsc

NC, NS, LANES = 2, 16, 16
NW = NC * NS


def _build_time_mats(wr, wi, T):
    tp = np.arange(T)[:, None].astype(np.float32)
    t = np.arange(T)[None, :].astype(np.float32)
    th = np.pi * (t - tp) / 2.0
    c = jnp.asarray(np.cos(th))
    s = jnp.asarray(np.sin(th))
    sign = jnp.asarray(((-1.0) ** (t + tp)).astype(np.float32))
    A = (wr[None, None, :, :, 0]
         + 2.0 * (c[:, :, None, None] * wr[None, None, :, :, 1]
                  - s[:, :, None, None] * wi[None, None, :, :, 1])
         + sign[:, :, None, None] * wr[None, None, :, :, 2]) / 4.0
    Ti, _, Cin, Cout = A.shape
    return jnp.transpose(A, (0, 2, 1, 3)).reshape(Ti * Cin, Ti * Cout)


def _prelude_body(h_ref, acat_ref, w1a_ref, w1b_ref, v_ref, bv_ref,
                  h2_ref, p_ref, q_ref, vn_ref):
    T = h_ref.shape[0]
    C = h_ref.shape[2]
    hcat = jnp.concatenate([h_ref[t] for t in range(T)], axis=-1)
    xh = jnp.dot(hcat, acat_ref[...], preferred_element_type=jnp.float32)
    h2 = hcat + jnp.where(xh > 0, xh, 0.2 * xh)
    h2_ref[...] = h2
    for t in range(T):
        blk = h2[:, t * C:(t + 1) * C]
        p_ref[t] = jnp.dot(blk, w1a_ref[...], preferred_element_type=jnp.float32)
        q_ref[t] = jnp.dot(blk, w1b_ref[...], preferred_element_type=jnp.float32)
    v = v_ref[...]
    vn_ref[...] = v + jnp.dot(v, bv_ref[...], preferred_element_type=jnp.float32)


def _prelude(h, acat, w1a, w1b, v, bv, T, BN, C, BR):
    nblk = BN // BR
    return pl.pallas_call(
        _prelude_body,
        grid=(nblk,),
        in_specs=[
            pl.BlockSpec((T, BR, C), lambda i: (0, i, 0)),
            pl.BlockSpec((T * C, T * C), lambda i: (0, 0)),
            pl.BlockSpec((C, C), lambda i: (0, 0)),
            pl.BlockSpec((C, C), lambda i: (0, 0)),
            pl.BlockSpec((BR, 3 * T), lambda i: (i, 0)),
            pl.BlockSpec((3 * T, 3 * T), lambda i: (0, 0)),
        ],
        out_specs=[
            pl.BlockSpec((BR, T * C), lambda i: (i, 0)),
            pl.BlockSpec((T, BR, C), lambda i: (0, i, 0)),
            pl.BlockSpec((T, BR, C), lambda i: (0, i, 0)),
            pl.BlockSpec((BR, 3 * T), lambda i: (i, 0)),
        ],
        out_shape=[
            jax.ShapeDtypeStruct((BN, T * C), jnp.float32),
            jax.ShapeDtypeStruct((T, BN, C), jnp.float32),
            jax.ShapeDtypeStruct((T, BN, C), jnp.float32),
            jax.ShapeDtypeStruct((BN, 3 * T), jnp.float32),
        ],
    )(h, acat, w1a, w1b, v, bv)


def _gather_kernel(TE, C, CE, tc_tiling=True):
    per_w = TE // NW
    nch = per_w // CE

    @functools.partial(
        pl.kernel,
        out_type=[jax.ShapeDtypeStruct((TE, C), jnp.float32),
                  jax.ShapeDtypeStruct((TE, C), jnp.float32)],
        compiler_params=None if tc_tiling else pltpu.CompilerParams(
            use_tc_tiling_on_sc=False),
        mesh=plsc.VectorSubcoreMesh(core_axis_name="c", subcore_axis_name="s"),
        scratch_types=[
            pltpu.VMEM((CE,), jnp.int32),
            pltpu.VMEM((CE,), jnp.int32),
            pltpu.VMEM((CE, C), jnp.float32),
            pltpu.VMEM((CE, C), jnp.float32),
            pltpu.SemaphoreType.DMA,
            pltpu.SemaphoreType.DMA,
        ],
    )
    def k(ptab, qtab, src_hbm, dst_hbm, gp_hbm, gq_hbm,
          sidx, didx, gpb, gqb, sem1, sem2):
        wid = lax.axis_index("c") * NS + lax.axis_index("s")
        wbase = wid * per_w

        def body(ci, _):
            base = wbase + ci * CE
            pltpu.sync_copy(src_hbm.at[pl.ds(base, CE)], sidx)
            pltpu.sync_copy(dst_hbm.at[pl.ds(base, CE)], didx)
            cp1 = pltpu.async_copy(ptab.at[sidx], gpb, sem1)
            cp2 = pltpu.async_copy(qtab.at[didx], gqb, sem2)
            cp1.wait()
            cp2.wait()
            pltpu.sync_copy(gpb, gp_hbm.at[pl.ds(base, CE)])
            pltpu.sync_copy(gqb, gq_hbm.at[pl.ds(base, CE)])
            return 0

        lax.fori_loop(0, nch, body, 0)

    return k


def _gather_add_kernel(TE, C, CE, dtype=jnp.float32):
    per_w = TE // NW
    nch = per_w // CE

    @functools.partial(
        pl.kernel,
        out_type=jax.ShapeDtypeStruct((TE, C), dtype),
        mesh=plsc.VectorSubcoreMesh(core_axis_name="c", subcore_axis_name="s"),
        scratch_types=[
            pltpu.VMEM((CE,), jnp.int32),
            pltpu.VMEM((CE,), jnp.int32),
            pltpu.VMEM((CE, C), dtype),
            pltpu.SemaphoreType.DMA,
        ],
    )
    def k(ptab, qtab, src_hbm, dst_hbm, g_hbm, sidx, didx, gb, sem):
        wid = lax.axis_index("c") * NS + lax.axis_index("s")
        wbase = wid * per_w

        def body(ci, _):
            base = wbase + ci * CE
            pltpu.sync_copy(src_hbm.at[pl.ds(base, CE)], sidx)
            pltpu.sync_copy(dst_hbm.at[pl.ds(base, CE)], didx)
            pltpu.async_copy(qtab.at[didx], gb, sem).wait()
            pltpu.async_copy(ptab.at[sidx], gb, sem, add=True).wait()
            pltpu.sync_copy(gb, g_hbm.at[pl.ds(base, CE)])
            return 0

        lax.fori_loop(0, nch, body, 0)

    return k


def _edge_body(g_ref, xs_ref, xd_ref, w1c_ref, b1_ref, we2_ref, b2_ref,
               wg_ref, bg_ref, out_ref):
    rel = xs_ref[...] - xd_ref[...]
    mask3 = lax.broadcasted_iota(jnp.int32, (1, rel.shape[1]), 1) < 3
    d2 = jnp.sum(jnp.where(mask3, rel * rel, 0.0), axis=-1, keepdims=True)
    g = (g_ref[...].astype(jnp.float32) + d2 * w1c_ref[...] + b1_ref[...])
    m1 = g * jax.nn.sigmoid(g)
    m2 = jnp.dot(m1, we2_ref[...], preferred_element_type=jnp.float32) + b2_ref[...]
    m2 = m2 * jax.nn.sigmoid(m2)
    gate = jax.nn.sigmoid(
        jnp.sum(m2 * wg_ref[...] + bg_ref[...], axis=-1, keepdims=True))
    out_ref[...] = m2 * gate


def _edge_mlp(g, xs, xd, w1c, b1, we2, b2, wgr, bgr, T, E, C, XW, BE):
    nblk = E // BE
    return pl.pallas_call(
        _edge_body,
        grid=(T, nblk),
        in_specs=[
            pl.BlockSpec((BE, C), lambda t, i: (t * nblk + i, 0)),
            pl.BlockSpec((BE, XW), lambda t, i: (i, 0)),
            pl.BlockSpec((BE, XW), lambda t, i: (i, 0)),
            pl.BlockSpec((1, C), lambda t, i: (0, 0)),
            pl.BlockSpec((1, C), lambda t, i: (0, 0)),
            pl.BlockSpec((C, C), lambda t, i: (0, 0)),
            pl.BlockSpec((1, C), lambda t, i: (0, 0)),
            pl.BlockSpec((1, C), lambda t, i: (0, 0)),
            pl.BlockSpec((1, C), lambda t, i: (0, 0)),
        ],
        out_specs=pl.BlockSpec((BE, C), lambda t, i: (t * nblk + i, 0)),
        out_shape=jax.ShapeDtypeStruct((T * E, C), jnp.float32),
    )(g, xs, xd, w1c, b1, we2, b2, wgr, bgr)


def _scatter_kernel(T, BN, E, C, CE):
    per_tile = E // NS
    nch = per_tile // CE
    rows = (BN // NS) // 8 * 8
    tail = BN - rows * NS
    reps = T // NC

    @functools.partial(
        pl.kernel,
        out_type=jax.ShapeDtypeStruct((T * BN, C), jnp.float32),
        mesh=plsc.VectorSubcoreMesh(core_axis_name="c", subcore_axis_name="s"),
        scratch_types=[
            pltpu.VMEM_SHARED((BN, C), jnp.float32),
            pltpu.VMEM((CE, C), jnp.float32),
            pltpu.VMEM((CE,), jnp.int32),
        ],
    )
    def k(m_hbm, dst_hbm, zero_hbm, agg_hbm, aggS, mbuf, dbuf):
        cid = lax.axis_index("c")
        sid = lax.axis_index("s")
        ebase0 = sid * per_tile

        for rep in range(reps):
            t = cid + NC * rep
            pltpu.sync_copy(zero_hbm.at[pl.ds(sid * rows, rows)],
                            aggS.at[pl.ds(sid * rows, rows)])
            if tail:
                @pl.when(sid == 0)
                def _():
                    pltpu.sync_copy(zero_hbm.at[pl.ds(NS * rows, tail)],
                                    aggS.at[pl.ds(NS * rows, tail)])
            plsc.subcore_barrier()

            def body(ci, _):
                base = t * E + ebase0 + ci * CE
                pltpu.sync_copy(m_hbm.at[pl.ds(base, CE)], mbuf)
                pltpu.sync_copy(dst_hbm.at[pl.ds(ebase0 + ci * CE, CE)], dbuf)
                pltpu.sync_copy(mbuf, aggS.at[dbuf], add=True)
                return 0

            lax.fori_loop(0, nch, body, 0)
            plsc.subcore_barrier()
            pltpu.sync_copy(aggS.at[pl.ds(sid * rows, rows)],
                            agg_hbm.at[pl.ds(t * BN + sid * rows, rows)])
            if tail:
                @pl.when(sid == 0)
                def _():
                    pltpu.sync_copy(aggS.at[pl.ds(NS * rows, tail)],
                                    agg_hbm.at[pl.ds(t * BN + NS * rows, tail)])

    return k


def _node_body(h2_ref, agg_ref, wa_ref, wb_ref, b1_ref, w2_ref, b2_ref, out_ref):
    u = (jnp.dot(h2_ref[...], wa_ref[...], preferred_element_type=jnp.float32)
         + jnp.dot(agg_ref[...], wb_ref[...], preferred_element_type=jnp.float32)
         + b1_ref[...])
    u = u * jax.nn.sigmoid(u)
    out_ref[...] = (h2_ref[...]
                    + jnp.dot(u, w2_ref[...], preferred_element_type=jnp.float32)
                    + b2_ref[...])


def _node_update(h2cat, agg, wa, wb, b1, w2, b2, T, BN, C, BR):
    nblk = BN // BR
    return pl.pallas_call(
        _node_body,
        grid=(T, nblk),
        in_specs=[
            pl.BlockSpec((BR, C), lambda t, i: (i, t)),
            pl.BlockSpec((BR, C), lambda t, i: (t * nblk + i, 0)),
            pl.BlockSpec((C, C), lambda t, i: (0, 0)),
            pl.BlockSpec((C, C), lambda t, i: (0, 0)),
            pl.BlockSpec((1, C), lambda t, i: (0, 0)),
            pl.BlockSpec((C, C), lambda t, i: (0, 0)),
            pl.BlockSpec((1, C), lambda t, i: (0, 0)),
        ],
        out_specs=pl.BlockSpec((BR, C), lambda t, i: (t * nblk + i, 0)),
        out_shape=jax.ShapeDtypeStruct((T * BN, C), jnp.float32),
    )(h2cat, agg, wa, wb, b1, w2, b2)


def kernel(h, x, vel_all, edge_index, tc_h_wr, tc_h_wi, tc_v_wr, tc_v_wi,
           W_e1, b_e1, W_e2, b_e2, W_g, b_g, W_n1, b_n1, W_n2, b_n2):
    T, BN, C = h.shape
    E = edge_index.shape[1]
    TE = T * E

    acat = _build_time_mats(tc_h_wr, tc_h_wi, T)
    a_v = _build_time_mats(tc_v_wr, tc_v_wi, T)
    bv = jnp.kron(a_v, jnp.eye(3, dtype=jnp.float32))
    w1a, w1b = W_e1[:C], W_e1[C:2 * C]
    w1c = W_e1[2 * C].reshape(1, C)
    src0 = edge_index[0].astype(jnp.int32)
    dst0 = edge_index[1].astype(jnp.int32)

    vflat = vel_all.reshape(BN, T * 3)
    h2cat, ptab, qtab, vnew = _prelude(h, acat, w1a, w1b, vflat, bv,
                                       T, BN, C, BR=1000)
    vel_out = vnew.reshape(BN, T, 3)

    XW = 16
    x16 = jnp.zeros((BN, XW), jnp.float32).at[:, :3].set(x)
    xs_g, xd_g = _gather_kernel(E, XW, CE=1000, tc_tiling=False)(
        x16, x16, src0, dst0)

    offs = jnp.repeat(jnp.arange(T, dtype=jnp.int32) * BN, E)
    src_all = jnp.tile(src0, T) + offs
    dst_all = jnp.tile(dst0, T) + offs
    g = _gather_add_kernel(TE, C, CE=400)(
        ptab.reshape(T * BN, C), qtab.reshape(T * BN, C), src_all, dst_all)

    bgr = jnp.full((1, C), b_g[0] / C, jnp.float32)
    m = _edge_mlp(g, xs_g, xd_g, w1c, b_e1.reshape(1, C), W_e2,
                  b_e2.reshape(1, C), W_g.reshape(1, C), bgr, T, E, C, XW, BE=2000)

    zeros = jnp.zeros((BN, C), jnp.float32)
    agg = _scatter_kernel(T, BN, E, C, CE=200)(m, dst0, zeros)

    h_out = _node_update(h2cat, agg, W_n1[:C], W_n1[C:], b_n1.reshape(1, C),
                         W_n2, b_n2.reshape(1, C), T, BN, C, BR=1000)
    return h_out.reshape(T, BN, C), vel_out
